# baseline (speedup 1.0000x reference)
import functools

import jax
import jax.numpy as jnp
from jax import lax
from jax.experimental import pallas as pl
from jax.experimental.pallas import tpu as pltpu
from jax.experimental.pallas import tpu_sc as plsc

_BATCH = 16384
_DIM = 128

_INFO = plsc.get_sparse_core_info()
_NC = _INFO.num_cores
_NS = _INFO.num_subcores
_NW = _NC * _NS
_BPW = _BATCH // _NW
_CHUNKS = (8, 40, 128, 128, 128, 80)
_OFFS = tuple(sum(_CHUNKS[:j]) for j in range(len(_CHUNKS)))
_NCHUNK = len(_CHUNKS)

_MESH = plsc.VectorSubcoreMesh(core_axis_name="c", subcore_axis_name="s")


@functools.partial(
    pl.kernel,
    mesh=_MESH,
    out_type=jax.ShapeDtypeStruct((_BATCH, _DIM), jnp.float32),
    scratch_types=[
        pltpu.VMEM((_BPW,), jnp.int32),
        pltpu.VMEM((_BPW, _DIM), jnp.float32),
        pltpu.VMEM_SHARED((1000, _DIM), jnp.float32),
        pltpu.SemaphoreType.DMA((_NCHUNK,)),
        pltpu.SemaphoreType.DMA,
        pltpu.SemaphoreType.DMA,
    ],
)
def _gather_rows(idx_hbm, table_hbm, out_hbm, idx_v, rows_v, table_sh, gsems, osem, isem):
    sid = lax.axis_index("s")
    wid = sid * _NC + lax.axis_index("c")
    base = wid * _BPW
    icopy = pltpu.async_copy(idx_hbm.at[pl.ds(base, _BPW)], idx_v, isem)

    @pl.when(sid == 0)
    def _stage():
        pltpu.sync_copy(table_hbm, table_sh)

    icopy.wait()
    plsc.subcore_barrier()
    gathers = [
        pltpu.async_copy(
            table_sh.at[idx_v.at[pl.ds(_OFFS[j], _CHUNKS[j])]],
            rows_v.at[pl.ds(_OFFS[j], _CHUNKS[j])],
            gsems.at[j],
        )
        for j in range(_NCHUNK)
    ]
    outs = []
    for j in range(_NCHUNK):
        gathers[j].wait()
        outs.append(
            pltpu.async_copy(
                rows_v.at[pl.ds(_OFFS[j], _CHUNKS[j])],
                out_hbm.at[pl.ds(base + _OFFS[j], _CHUNKS[j])],
                osem,
            )
        )
    for o in outs:
        o.wait()


def kernel(noise_levels, table):
    idx = noise_levels.astype(jnp.int32)
    return _gather_rows(idx, table)


# speedup vs baseline: 1.0070x; 1.0070x over previous
import functools

import jax
import jax.numpy as jnp
from jax import lax
from jax.experimental import pallas as pl
from jax.experimental.pallas import tpu as pltpu
from jax.experimental.pallas import tpu_sc as plsc

_BATCH = 16384
_DIM = 128

_INFO = plsc.get_sparse_core_info()
_NC = _INFO.num_cores
_NS = _INFO.num_subcores
_NW = _NC * _NS
_BPW = _BATCH // _NW
_CHUNKS = (8, 24, 48, 96, 128, 128, 56, 24)
_OFFS = tuple(sum(_CHUNKS[:j]) for j in range(len(_CHUNKS)))
_NCHUNK = len(_CHUNKS)

_MESH = plsc.VectorSubcoreMesh(core_axis_name="c", subcore_axis_name="s")


@functools.partial(
    pl.kernel,
    mesh=_MESH,
    out_type=jax.ShapeDtypeStruct((_BATCH, _DIM), jnp.float32),
    scratch_types=[
        pltpu.VMEM((_BPW,), jnp.int32),
        pltpu.VMEM((_BPW, _DIM), jnp.float32),
        pltpu.VMEM_SHARED((1000, _DIM), jnp.float32),
        pltpu.SemaphoreType.DMA((_NCHUNK,)),
        pltpu.SemaphoreType.DMA,
        pltpu.SemaphoreType.DMA,
    ],
)
def _gather_rows(idx_hbm, table_hbm, out_hbm, idx_v, rows_v, table_sh, gsems, osem, isem):
    sid = lax.axis_index("s")
    wid = sid * _NC + lax.axis_index("c")
    base = wid * _BPW
    icopy = pltpu.async_copy(idx_hbm.at[pl.ds(base, _BPW)], idx_v, isem)

    @pl.when(sid == 0)
    def _stage():
        pltpu.sync_copy(table_hbm, table_sh)

    icopy.wait()
    plsc.subcore_barrier()
    gathers = [
        pltpu.async_copy(
            table_sh.at[idx_v.at[pl.ds(_OFFS[j], _CHUNKS[j])]],
            rows_v.at[pl.ds(_OFFS[j], _CHUNKS[j])],
            gsems.at[j],
        )
        for j in range(_NCHUNK)
    ]
    outs = []
    for j in range(_NCHUNK):
        gathers[j].wait()
        outs.append(
            pltpu.async_copy(
                rows_v.at[pl.ds(_OFFS[j], _CHUNKS[j])],
                out_hbm.at[pl.ds(base + _OFFS[j], _CHUNKS[j])],
                osem,
            )
        )
    for o in outs:
        o.wait()


def kernel(noise_levels, table):
    idx = noise_levels.astype(jnp.int32)
    return _gather_rows(idx, table)
